# Initial kernel scaffold; baseline (speedup 1.0000x reference)
import jax
import jax.numpy as jnp
from jax.experimental import pallas as pl


def kernel(hidden_state, corr_index, nei_index, W_att, b_att):
    raise NotImplementedError("write your pallas kernel here")


# speedup vs baseline: 19.4105x; 19.4105x over previous
import jax
import jax.numpy as jnp
from jax.experimental import pallas as pl


def _social_kernel(h_ref, nei_ref, w_ref, b_ref, out_ref):
    h = h_ref[:]
    w = w_ref[:]
    m_dim = h.shape[1]
    w1 = w[:, :m_dim]
    w2 = w[:, m_dim:]
    a = jax.lax.dot_general(h, w1, (((1,), (1,)), ((), ())),
                            preferred_element_type=jnp.float32)
    crow = jax.lax.dot_general(w2, h, (((1,), (1,)), ((), ())),
                               preferred_element_type=jnp.float32)
    logits = a + crow + b_ref[0, 0]
    mask = nei_ref[:] > 0
    logits = jnp.where(mask, logits, 0.0)
    logits = jnp.where(logits == 0.0, -1e-6, logits)
    mx = jnp.max(logits, axis=1, keepdims=True)
    e = jnp.exp(logits - mx)
    p = e / jnp.sum(e, axis=1, keepdims=True)
    p = jnp.where(mask, p, 0.0)
    out_ref[:] = jnp.dot(p, h, preferred_element_type=jnp.float32)


def kernel(hidden_state, corr_index, nei_index, W_att, b_att):
    n, m_dim = hidden_state.shape
    return pl.pallas_call(
        _social_kernel,
        out_shape=jax.ShapeDtypeStruct((n, m_dim), jnp.float32),
    )(hidden_state, nei_index, W_att, b_att.reshape(1, 1))
